# Initial kernel scaffold; baseline (speedup 1.0000x reference)
import jax
import jax.numpy as jnp
from jax.experimental import pallas as pl


def kernel(sinogram, v0, d_delay, ring_error):
    raise NotImplementedError("write your pallas kernel here")


# speedup vs baseline: 209.5975x; 209.5975x over previous
import functools

import jax
import jax.numpy as jnp
from jax import lax
from jax.experimental import pallas as pl
from jax.experimental.pallas import tpu as pltpu
from jax.experimental.pallas import tpu_sc as plsc

R_RING = 0.05
N_TR = 512
T_SAMPLE = 3.75e-05
N_TIME = 2048
GRID = 200
NPIX = GRID * GRID
NPIX_PAD = 40960
NC = 2
NS = 16
NW = NC * NS
T_PER_W = N_TR // NW
CHUNK = 4000
NCHUNK = NPIX // CHUNK
GPC = CHUNK // 16
STRIPE = NPIX_PAD // NS
SGROUPS = STRIPE // 16

_MAGIC = 8388608.0


def _dist_flat():
    x_vec = (-0.02 + 0.0002 * jnp.arange(GRID, dtype=jnp.float32)).reshape(1, -1, 1)
    y_vec = (-0.02 + 0.0002 * jnp.arange(GRID, dtype=jnp.float32)).reshape(1, 1, -1)
    angle = (2.0 * jnp.pi / N_TR) * (jnp.arange(N_TR, dtype=jnp.float32) + 1.0)
    angle = angle.reshape(-1, 1, 1)
    x_t = R_RING * jnp.cos(angle - jnp.pi)
    y_t = R_RING * jnp.sin(angle - jnp.pi)
    return jnp.sqrt((x_t - x_vec) ** 2 + (y_t - y_vec) ** 2).reshape(-1)


_mesh = plsc.VectorSubcoreMesh(core_axis_name="c", subcore_axis_name="s")


@functools.partial(
    pl.kernel,
    out_type=jax.ShapeDtypeStruct((NC * NPIX_PAD,), jnp.float32),
    mesh=_mesh,
    scratch_types=[
        pltpu.VMEM((NPIX_PAD,), jnp.float32),
        pltpu.VMEM((N_TIME,), jnp.float32),
        pltpu.VMEM((CHUNK,), jnp.float32),
        pltpu.VMEM((STRIPE,), jnp.float32),
        pltpu.VMEM((STRIPE,), jnp.float32),
        pltpu.VMEM((16,), jnp.float32),
        pltpu.VMEM_SHARED((NS * NPIX_PAD,), jnp.float32),
    ],
    compiler_params=pltpu.CompilerParams(needs_layout_passes=False),
)
def _das_sc(sino_hbm, dist_hbm, scal_hbm, out_hbm,
            acc_v, row_v, dist_v, sin_v, sacc_v, scal_v, shared):
    c = lax.axis_index("c")
    s = lax.axis_index("s")
    w = c * NS + s

    pltpu.sync_copy(scal_hbm, scal_v)
    sv = scal_v[pl.ds(0, 16)]
    v0 = sv[0]
    dd = sv[1]
    re = sv[2]
    vts_b = jnp.full((16,), v0 * jnp.float32(T_SAMPLE), jnp.float32)
    re_b = jnp.full((16,), re, jnp.float32)
    dd_b = jnp.full((16,), dd, jnp.float32)

    lane = lax.iota(jnp.int32, 16)
    head_mask = jnp.where(lane == 0, jnp.float32(0), jnp.float32(1))
    tail_mask = jnp.where(lane == 15, jnp.float32(0), jnp.float32(1))
    zero16 = jnp.zeros((16,), jnp.float32)

    def zacc(i, carry):
        acc_v[pl.ds(i * 16, 16)] = zero16
        return carry

    lax.fori_loop(0, NPIX_PAD // 16, zacc, 0)

    t0 = w * T_PER_W

    def t_body(jt, carry):
        t = t0 + jt
        pltpu.sync_copy(sino_hbm.at[pl.ds(t * N_TIME, N_TIME)], row_v)
        row_v[pl.ds(0, 16)] = row_v[pl.ds(0, 16)] * head_mask
        row_v[pl.ds(N_TIME - 16, 16)] = row_v[pl.ds(N_TIME - 16, 16)] * tail_mask

        def c_body(cc, carry2):
            pltpu.sync_copy(dist_hbm.at[pl.ds(t * NPIX + cc * CHUNK, CHUNK)],
                            dist_v)
            base = cc * CHUNK

            def g_body(g, carry3):
                off = g * 16
                d = dist_v[pl.ds(off, 16)]
                q = ((d + re_b) - dd_b) / vts_b
                r = (q + _MAGIC) - _MAGIC
                idx = r.astype(jnp.int32)
                idx = jnp.minimum(jnp.maximum(idx, 0), N_TIME - 1)
                vals = plsc.load_gather(row_v, [idx])
                aoff = base + off
                acc_v[pl.ds(aoff, 16)] = acc_v[pl.ds(aoff, 16)] + vals
                return carry3

            lax.fori_loop(0, GPC, g_body, 0)
            return carry2

        lax.fori_loop(0, NCHUNK, c_body, 0)
        return carry

    lax.fori_loop(0, T_PER_W, t_body, 0)

    pltpu.sync_copy(acc_v, shared.at[pl.ds(s * NPIX_PAD, NPIX_PAD)])
    plsc.subcore_barrier()

    sbase = s * STRIPE
    pltpu.sync_copy(shared.at[pl.ds(sbase, STRIPE)], sacc_v)

    def r_body(t2, carry):
        pltpu.sync_copy(shared.at[pl.ds(t2 * NPIX_PAD + sbase, STRIPE)], sin_v)

        def a_body(g, carry2):
            off = g * 16
            sacc_v[pl.ds(off, 16)] = sacc_v[pl.ds(off, 16)] + sin_v[pl.ds(off, 16)]
            return carry2

        lax.fori_loop(0, SGROUPS, a_body, 0)
        return carry

    lax.fori_loop(1, NS, r_body, 0)

    scale = jnp.full((16,), jnp.float32(1.0 / N_TR), jnp.float32)

    def s_body(g, carry):
        off = g * 16
        sacc_v[pl.ds(off, 16)] = sacc_v[pl.ds(off, 16)] * scale
        return carry

    lax.fori_loop(0, SGROUPS, s_body, 0)
    pltpu.sync_copy(sacc_v, out_hbm.at[pl.ds(c * NPIX_PAD + sbase, STRIPE)])


def kernel(sinogram, v0, d_delay, ring_error):
    dist = _dist_flat()
    scal = jnp.concatenate([
        v0.astype(jnp.float32),
        d_delay.astype(jnp.float32),
        ring_error.astype(jnp.float32),
        jnp.zeros((13,), jnp.float32),
    ])
    part = _das_sc(sinogram.reshape(-1), dist, scal)
    out = part[:NPIX_PAD] + part[NPIX_PAD:]
    return out[:NPIX].reshape(GRID, GRID)


# speedup vs baseline: 354.5383x; 1.6915x over previous
import functools

import jax
import jax.numpy as jnp
from jax import lax
from jax.experimental import pallas as pl
from jax.experimental.pallas import tpu as pltpu
from jax.experimental.pallas import tpu_sc as plsc

R_RING = 0.05
N_TR = 512
T_SAMPLE = 3.75e-05
N_TIME = 2048
GRID = 200
NPIX = GRID * GRID
NPIX_PAD = 40960
NC = 2
NS = 16
NW = NC * NS
T_PER_W = N_TR // NW
CHUNK = 4000
NCHUNK = NPIX // CHUNK
GPC = CHUNK // 16
UNROLL = 5
STRIPE = NPIX_PAD // NS
SGROUPS = STRIPE // 16

_MAGIC = 8388608.0


def _dist_flat():
    x_vec = (-0.02 + 0.0002 * jnp.arange(GRID, dtype=jnp.float32)).reshape(1, -1, 1)
    y_vec = (-0.02 + 0.0002 * jnp.arange(GRID, dtype=jnp.float32)).reshape(1, 1, -1)
    angle = (2.0 * jnp.pi / N_TR) * (jnp.arange(N_TR, dtype=jnp.float32) + 1.0)
    angle = angle.reshape(-1, 1, 1)
    x_t = R_RING * jnp.cos(angle - jnp.pi)
    y_t = R_RING * jnp.sin(angle - jnp.pi)
    return jnp.sqrt((x_t - x_vec) ** 2 + (y_t - y_vec) ** 2).reshape(-1)


_mesh = plsc.VectorSubcoreMesh(core_axis_name="c", subcore_axis_name="s")


@functools.partial(
    pl.kernel,
    out_type=jax.ShapeDtypeStruct((NC * NPIX_PAD,), jnp.float32),
    mesh=_mesh,
    scratch_types=[
        pltpu.VMEM((NPIX_PAD,), jnp.float32),
        pltpu.VMEM((N_TIME,), jnp.float32),
        pltpu.VMEM((CHUNK,), jnp.float32),
        pltpu.VMEM((CHUNK,), jnp.float32),
        pltpu.VMEM((STRIPE,), jnp.float32),
        pltpu.VMEM((STRIPE,), jnp.float32),
        pltpu.VMEM((16,), jnp.float32),
        pltpu.VMEM_SHARED((NS * NPIX_PAD,), jnp.float32),
        pltpu.SemaphoreType.DMA,
        pltpu.SemaphoreType.DMA,
    ],
    compiler_params=pltpu.CompilerParams(needs_layout_passes=False),
)
def _das_sc(sino_hbm, dist_hbm, scal_hbm, out_hbm,
            acc_v, row_v, dist_a, dist_b, sin_v, sacc_v, scal_v, shared,
            sem_a, sem_b):
    c = lax.axis_index("c")
    s = lax.axis_index("s")
    w = c * NS + s

    pltpu.sync_copy(scal_hbm, scal_v)
    sv = scal_v[pl.ds(0, 16)]
    v0 = sv[0]
    dd = sv[1]
    re = sv[2]
    vts_b = jnp.full((16,), v0 * jnp.float32(T_SAMPLE), jnp.float32)
    re_b = jnp.full((16,), re, jnp.float32)
    dd_b = jnp.full((16,), dd, jnp.float32)

    lane = lax.iota(jnp.int32, 16)
    head_mask = jnp.where(lane == 0, jnp.float32(0), jnp.float32(1))
    tail_mask = jnp.where(lane == 15, jnp.float32(0), jnp.float32(1))
    zero16 = jnp.zeros((16,), jnp.float32)

    def zacc(i, carry):
        base = i * 128
        for u in range(8):
            acc_v[pl.ds(base + u * 16, 16)] = zero16
        return carry

    lax.fori_loop(0, NPIX_PAD // 128, zacc, 0)

    t0 = w * T_PER_W

    def _gather_groups(dist_v, base, i):
        off0 = i * (16 * UNROLL)
        for u in range(UNROLL):
            off = off0 + u * 16
            d = dist_v[pl.ds(off, 16)]
            q = ((d + re_b) - dd_b) / vts_b
            r = (q + _MAGIC) - _MAGIC
            idx = r.astype(jnp.int32)
            idx = jnp.minimum(jnp.maximum(idx, 0), N_TIME - 1)
            vals = plsc.load_gather(row_v, [idx])
            aoff = base + off
            acc_v[pl.ds(aoff, 16)] = acc_v[pl.ds(aoff, 16)] + vals

    def t_body(jt, carry):
        t = t0 + jt
        pltpu.sync_copy(sino_hbm.at[pl.ds(t * N_TIME, N_TIME)], row_v)
        row_v[pl.ds(0, 16)] = row_v[pl.ds(0, 16)] * head_mask
        row_v[pl.ds(N_TIME - 16, 16)] = row_v[pl.ds(N_TIME - 16, 16)] * tail_mask

        bufs = (dist_a, dist_b)
        sems = (sem_a, sem_b)
        dbase = t * NPIX
        pend = [pltpu.async_copy(dist_hbm.at[pl.ds(dbase, CHUNK)],
                                 bufs[0], sems[0])]

        for cc in range(NCHUNK):
            if cc + 1 < NCHUNK:
                nxt = (cc + 1) % 2
                pend.append(pltpu.async_copy(
                    dist_hbm.at[pl.ds(dbase + (cc + 1) * CHUNK, CHUNK)],
                    bufs[nxt], sems[nxt]))
            pend[cc].wait()
            cur = bufs[cc % 2]
            base = cc * CHUNK

            def c_body(i, carry2, cur=cur, base=base):
                _gather_groups(cur, base, i)
                return carry2

            lax.fori_loop(0, GPC // UNROLL, c_body, 0)
        return carry

    lax.fori_loop(0, T_PER_W, t_body, 0)

    pltpu.sync_copy(acc_v, shared.at[pl.ds(s * NPIX_PAD, NPIX_PAD)])
    plsc.subcore_barrier()

    sbase = s * STRIPE
    pltpu.sync_copy(shared.at[pl.ds(sbase, STRIPE)], sacc_v)

    def r_body(t2, carry):
        pltpu.sync_copy(shared.at[pl.ds(t2 * NPIX_PAD + sbase, STRIPE)], sin_v)

        def a_body(g, carry2):
            base = g * 128
            for u in range(8):
                off = base + u * 16
                sacc_v[pl.ds(off, 16)] = (sacc_v[pl.ds(off, 16)]
                                          + sin_v[pl.ds(off, 16)])
            return carry2

        lax.fori_loop(0, SGROUPS // 8, a_body, 0)
        return carry

    lax.fori_loop(1, NS, r_body, 0)

    scale = jnp.full((16,), jnp.float32(1.0 / N_TR), jnp.float32)

    def s_body(g, carry):
        base = g * 128
        for u in range(8):
            off = base + u * 16
            sacc_v[pl.ds(off, 16)] = sacc_v[pl.ds(off, 16)] * scale
        return carry

    lax.fori_loop(0, SGROUPS // 8, s_body, 0)
    pltpu.sync_copy(sacc_v, out_hbm.at[pl.ds(c * NPIX_PAD + sbase, STRIPE)])


def kernel(sinogram, v0, d_delay, ring_error):
    dist = _dist_flat()
    scal = jnp.concatenate([
        v0.astype(jnp.float32),
        d_delay.astype(jnp.float32),
        ring_error.astype(jnp.float32),
        jnp.zeros((13,), jnp.float32),
    ])
    part = _das_sc(sinogram.reshape(-1), dist, scal)
    out = part[:NPIX_PAD] + part[NPIX_PAD:]
    return out[:NPIX].reshape(GRID, GRID)


# speedup vs baseline: 604.3220x; 1.7045x over previous
import functools

import jax
import jax.numpy as jnp
from jax import lax
from jax.experimental import pallas as pl
from jax.experimental.pallas import tpu as pltpu
from jax.experimental.pallas import tpu_sc as plsc

R_RING = 0.05
N_TR = 512
T_SAMPLE = 3.75e-05
N_TIME = 2048
GRID = 200
NPIX = GRID * GRID
NPIX_PAD = 40960
BROWS = NPIX_PAD // 128
NC = 2
NS = 16
NW = NC * NS
T_PER_W = N_TR // NW
CHUNK = 4096
NCHUNK = NPIX_PAD // CHUNK
GPC = CHUNK // 16
UNROLL = 4
STRIPE = NPIX_PAD // NS
SGROUPS = STRIPE // 16

_MAGIC = 8388608.0


def _idx_tc_kernel(xt_ref, yt_ref, scal_ref, out_ref):
    t = pl.program_id(0)
    xt = xt_ref[t]
    yt = yt_ref[t]
    v0 = scal_ref[0]
    dd = scal_ref[1]
    re = scal_ref[2]
    vts = v0 * jnp.float32(T_SAMPLE)

    i = lax.broadcasted_iota(jnp.int32, (BROWS, 128), 0)
    j = lax.broadcasted_iota(jnp.int32, (BROWS, 128), 1)
    p = i * 128 + j
    px = (p // GRID).astype(jnp.float32)
    py = (p % GRID).astype(jnp.float32)
    x = jnp.float32(-0.02) + jnp.float32(0.0002) * px
    y = jnp.float32(-0.02) + jnp.float32(0.0002) * py
    dx = xt - x
    dy = yt - y
    dist = jnp.sqrt(dx * dx + dy * dy)
    q = ((dist + re) - dd) / vts
    r = (q + _MAGIC) - _MAGIC
    r = jnp.minimum(jnp.maximum(r, jnp.float32(0.0)),
                    jnp.float32(N_TIME - 1))
    out_ref[0] = r.astype(jnp.int32)


def _compute_indices(xt, yt, scal):
    return pl.pallas_call(
        _idx_tc_kernel,
        grid=(N_TR,),
        in_specs=[
            pl.BlockSpec(memory_space=pltpu.SMEM),
            pl.BlockSpec(memory_space=pltpu.SMEM),
            pl.BlockSpec(memory_space=pltpu.SMEM),
        ],
        out_specs=pl.BlockSpec((1, BROWS, 128), lambda t: (t, 0, 0)),
        out_shape=jax.ShapeDtypeStruct((N_TR, BROWS, 128), jnp.int32),
    )(xt, yt, scal)


_mesh = plsc.VectorSubcoreMesh(core_axis_name="c", subcore_axis_name="s")


@functools.partial(
    pl.kernel,
    out_type=jax.ShapeDtypeStruct((NC * NPIX_PAD,), jnp.float32),
    mesh=_mesh,
    scratch_types=[
        pltpu.VMEM((NPIX_PAD,), jnp.float32),
        pltpu.VMEM((N_TIME,), jnp.float32),
        pltpu.VMEM((CHUNK,), jnp.int32),
        pltpu.VMEM((CHUNK,), jnp.int32),
        pltpu.VMEM((STRIPE,), jnp.float32),
        pltpu.VMEM((STRIPE,), jnp.float32),
        pltpu.VMEM_SHARED((NS * NPIX_PAD,), jnp.float32),
        pltpu.SemaphoreType.DMA,
        pltpu.SemaphoreType.DMA,
    ],
    compiler_params=pltpu.CompilerParams(needs_layout_passes=False),
)
def _das_sc(sino_hbm, idx_hbm, out_hbm,
            acc_v, rows_v, idx_a, idx_b, sin_v, sacc_v, shared,
            sem_a, sem_b):
    c = lax.axis_index("c")
    s = lax.axis_index("s")
    w = c * NS + s

    lane = lax.iota(jnp.int32, 16)
    head_mask = jnp.where(lane == 0, jnp.float32(0), jnp.float32(1))
    tail_mask = jnp.where(lane == 15, jnp.float32(0), jnp.float32(1))
    zero16 = jnp.zeros((16,), jnp.float32)

    def zacc(i, carry):
        base = i * 128
        for u in range(8):
            acc_v[pl.ds(base + u * 16, 16)] = zero16
        return carry

    lax.fori_loop(0, NPIX_PAD // 128, zacc, 0)

    t0 = w * T_PER_W

    def _gather_chunk(idx_v, row_ref, base):
        @plsc.parallel_loop(0, GPC, step=1, unroll=UNROLL)
        def _(g):
            off = g * 16
            idx = idx_v[pl.ds(off, 16)]
            vals = plsc.load_gather(row_ref, [idx])
            aoff = base + off
            acc_v[pl.ds(aoff, 16)] = acc_v[pl.ds(aoff, 16)] + vals

    def t_body(jt, carry):
        t = t0 + jt
        pltpu.sync_copy(sino_hbm.at[pl.ds(t * N_TIME, N_TIME)], rows_v)
        rows_v[pl.ds(0, 16)] = rows_v[pl.ds(0, 16)] * head_mask
        rows_v[pl.ds(N_TIME - 16, 16)] = (
            rows_v[pl.ds(N_TIME - 16, 16)] * tail_mask)

        bufs = (idx_a, idx_b)
        sems = (sem_a, sem_b)
        dbase = t * NPIX_PAD
        pend = [pltpu.async_copy(idx_hbm.at[pl.ds(dbase, CHUNK)],
                                 bufs[0], sems[0])]

        for cc in range(NCHUNK):
            if cc + 1 < NCHUNK:
                nxt = (cc + 1) % 2
                pend.append(pltpu.async_copy(
                    idx_hbm.at[pl.ds(dbase + (cc + 1) * CHUNK, CHUNK)],
                    bufs[nxt], sems[nxt]))
            pend[cc].wait()
            _gather_chunk(bufs[cc % 2], rows_v, cc * CHUNK)
        return carry

    lax.fori_loop(0, T_PER_W, t_body, 0)

    pltpu.sync_copy(acc_v, shared.at[pl.ds(s * NPIX_PAD, NPIX_PAD)])
    plsc.subcore_barrier()

    sbase = s * STRIPE
    pltpu.sync_copy(shared.at[pl.ds(sbase, STRIPE)], sacc_v)

    def r_body(t2, carry):
        pltpu.sync_copy(shared.at[pl.ds(t2 * NPIX_PAD + sbase, STRIPE)], sin_v)

        def a_body(g, carry2):
            base = g * 128
            for u in range(8):
                off = base + u * 16
                sacc_v[pl.ds(off, 16)] = (sacc_v[pl.ds(off, 16)]
                                          + sin_v[pl.ds(off, 16)])
            return carry2

        lax.fori_loop(0, SGROUPS // 8, a_body, 0)
        return carry

    lax.fori_loop(1, NS, r_body, 0)

    scale = jnp.full((16,), jnp.float32(1.0 / N_TR), jnp.float32)

    def s_body(g, carry):
        base = g * 128
        for u in range(8):
            off = base + u * 16
            sacc_v[pl.ds(off, 16)] = sacc_v[pl.ds(off, 16)] * scale
        return carry

    lax.fori_loop(0, SGROUPS // 8, s_body, 0)
    pltpu.sync_copy(sacc_v, out_hbm.at[pl.ds(c * NPIX_PAD + sbase, STRIPE)])


def kernel(sinogram, v0, d_delay, ring_error):
    angle = (2.0 * jnp.pi / N_TR) * (jnp.arange(N_TR, dtype=jnp.float32) + 1.0)
    x_t = R_RING * jnp.cos(angle - jnp.pi)
    y_t = R_RING * jnp.sin(angle - jnp.pi)
    scal = jnp.concatenate([
        v0.astype(jnp.float32),
        d_delay.astype(jnp.float32),
        ring_error.astype(jnp.float32),
        jnp.zeros((13,), jnp.float32),
    ])
    idx = _compute_indices(x_t, y_t, scal).reshape(-1)
    part = _das_sc(sinogram.reshape(-1), idx)
    out = part[:NPIX_PAD] + part[NPIX_PAD:]
    return out[:NPIX].reshape(GRID, GRID)


# speedup vs baseline: 1094.7325x; 1.8115x over previous
import functools

import jax
import jax.numpy as jnp
from jax import lax
from jax.experimental import pallas as pl
from jax.experimental.pallas import tpu as pltpu
from jax.experimental.pallas import tpu_sc as plsc

R_RING = 0.05
N_TR = 512
T_SAMPLE = 3.75e-05
N_TIME = 2048
GRID = 200
NPIX = GRID * GRID
NPIX_PAD = 40960
BROWS = NPIX_PAD // 128
NC = 2
NS = 16
NW = NC * NS
T_PER_W = N_TR // NW
CHUNK = 4096
NCHUNK = NPIX_PAD // CHUNK
GPC = CHUNK // 16
UNROLL = 4
STRIPE = NPIX_PAD // NS
SGROUPS = STRIPE // 16

_MAGIC = 8388608.0


T_BLK = 8


def _idx_tc_kernel(xt_ref, yt_ref, scal_ref, xmap_ref, ymap_ref, out_ref):
    pid = pl.program_id(0)
    v0 = scal_ref[0]
    dd = scal_ref[1]
    re = scal_ref[2]
    vts = v0 * jnp.float32(T_SAMPLE)
    x = xmap_ref[...]
    y = ymap_ref[...]
    for r in range(T_BLK):
        t = pid * T_BLK + r
        dx = xt_ref[t] - x
        dy = yt_ref[t] - y
        dist = jnp.sqrt(dx * dx + dy * dy)
        q = ((dist + re) - dd) / vts
        rr = (q + _MAGIC) - _MAGIC
        rr = jnp.minimum(jnp.maximum(rr, jnp.float32(0.0)),
                         jnp.float32(N_TIME - 1))
        out_ref[r] = rr.astype(jnp.int32)


def _compute_indices(xt, yt, scal, xmap, ymap):
    return pl.pallas_call(
        _idx_tc_kernel,
        grid=(N_TR // T_BLK,),
        in_specs=[
            pl.BlockSpec(memory_space=pltpu.SMEM),
            pl.BlockSpec(memory_space=pltpu.SMEM),
            pl.BlockSpec(memory_space=pltpu.SMEM),
            pl.BlockSpec((BROWS, 128), lambda g: (0, 0)),
            pl.BlockSpec((BROWS, 128), lambda g: (0, 0)),
        ],
        out_specs=pl.BlockSpec((T_BLK, BROWS, 128), lambda g: (g, 0, 0)),
        out_shape=jax.ShapeDtypeStruct((N_TR, BROWS, 128), jnp.int32),
    )(xt, yt, scal, xmap, ymap)


_mesh = plsc.VectorSubcoreMesh(core_axis_name="c", subcore_axis_name="s")


@functools.partial(
    pl.kernel,
    out_type=jax.ShapeDtypeStruct((NC * NPIX_PAD,), jnp.float32),
    mesh=_mesh,
    scratch_types=[
        pltpu.VMEM((NPIX_PAD,), jnp.float32),
        pltpu.VMEM((N_TIME,), jnp.float32),
        pltpu.VMEM((CHUNK,), jnp.int32),
        pltpu.VMEM((CHUNK,), jnp.int32),
        pltpu.VMEM((STRIPE,), jnp.float32),
        pltpu.VMEM((STRIPE,), jnp.float32),
        pltpu.VMEM_SHARED((NS * NPIX_PAD,), jnp.float32),
        pltpu.SemaphoreType.DMA,
        pltpu.SemaphoreType.DMA,
    ],
    compiler_params=pltpu.CompilerParams(needs_layout_passes=False),
)
def _das_sc(sino_hbm, idx_hbm, out_hbm,
            acc_v, rows_v, idx_a, idx_b, sin_v, sacc_v, shared,
            sem_a, sem_b):
    c = lax.axis_index("c")
    s = lax.axis_index("s")
    w = c * NS + s

    lane = lax.iota(jnp.int32, 16)
    head_mask = jnp.where(lane == 0, jnp.float32(0), jnp.float32(1))
    tail_mask = jnp.where(lane == 15, jnp.float32(0), jnp.float32(1))
    zero16 = jnp.zeros((16,), jnp.float32)

    def zacc(i, carry):
        base = i * 128
        for u in range(8):
            acc_v[pl.ds(base + u * 16, 16)] = zero16
        return carry

    lax.fori_loop(0, NPIX_PAD // 128, zacc, 0)

    t0 = w * T_PER_W

    def _gather_chunk(idx_v, row_ref, base):
        @plsc.parallel_loop(0, GPC, step=1, unroll=UNROLL)
        def _(g):
            off = g * 16
            idx = idx_v[pl.ds(off, 16)]
            vals = plsc.load_gather(row_ref, [idx])
            aoff = base + off
            acc_v[pl.ds(aoff, 16)] = acc_v[pl.ds(aoff, 16)] + vals

    def t_body(jt, carry):
        t = t0 + jt
        pltpu.sync_copy(sino_hbm.at[pl.ds(t * N_TIME, N_TIME)], rows_v)
        rows_v[pl.ds(0, 16)] = rows_v[pl.ds(0, 16)] * head_mask
        rows_v[pl.ds(N_TIME - 16, 16)] = (
            rows_v[pl.ds(N_TIME - 16, 16)] * tail_mask)

        bufs = (idx_a, idx_b)
        sems = (sem_a, sem_b)
        dbase = t * NPIX_PAD
        pend = [pltpu.async_copy(idx_hbm.at[pl.ds(dbase, CHUNK)],
                                 bufs[0], sems[0])]

        for cc in range(NCHUNK):
            if cc + 1 < NCHUNK:
                nxt = (cc + 1) % 2
                pend.append(pltpu.async_copy(
                    idx_hbm.at[pl.ds(dbase + (cc + 1) * CHUNK, CHUNK)],
                    bufs[nxt], sems[nxt]))
            pend[cc].wait()
            _gather_chunk(bufs[cc % 2], rows_v, cc * CHUNK)
        return carry

    lax.fori_loop(0, T_PER_W, t_body, 0)

    pltpu.sync_copy(acc_v, shared.at[pl.ds(s * NPIX_PAD, NPIX_PAD)])
    plsc.subcore_barrier()

    sbase = s * STRIPE
    pltpu.sync_copy(shared.at[pl.ds(sbase, STRIPE)], sacc_v)

    def r_body(t2, carry):
        pltpu.sync_copy(shared.at[pl.ds(t2 * NPIX_PAD + sbase, STRIPE)], sin_v)

        def a_body(g, carry2):
            base = g * 128
            for u in range(8):
                off = base + u * 16
                sacc_v[pl.ds(off, 16)] = (sacc_v[pl.ds(off, 16)]
                                          + sin_v[pl.ds(off, 16)])
            return carry2

        lax.fori_loop(0, SGROUPS // 8, a_body, 0)
        return carry

    lax.fori_loop(1, NS, r_body, 0)

    scale = jnp.full((16,), jnp.float32(1.0 / N_TR), jnp.float32)

    def s_body(g, carry):
        base = g * 128
        for u in range(8):
            off = base + u * 16
            sacc_v[pl.ds(off, 16)] = sacc_v[pl.ds(off, 16)] * scale
        return carry

    lax.fori_loop(0, SGROUPS // 8, s_body, 0)
    pltpu.sync_copy(sacc_v, out_hbm.at[pl.ds(c * NPIX_PAD + sbase, STRIPE)])


def kernel(sinogram, v0, d_delay, ring_error):
    angle = (2.0 * jnp.pi / N_TR) * (jnp.arange(N_TR, dtype=jnp.float32) + 1.0)
    x_t = R_RING * jnp.cos(angle - jnp.pi)
    y_t = R_RING * jnp.sin(angle - jnp.pi)
    p = jnp.arange(NPIX_PAD, dtype=jnp.int32)
    xmap = (jnp.float32(-0.02)
            + jnp.float32(0.0002) * (p // GRID).astype(jnp.float32))
    ymap = (jnp.float32(-0.02)
            + jnp.float32(0.0002) * (p % GRID).astype(jnp.float32))
    xmap = xmap.reshape(BROWS, 128)
    ymap = ymap.reshape(BROWS, 128)
    scal = jnp.concatenate([
        v0.astype(jnp.float32),
        d_delay.astype(jnp.float32),
        ring_error.astype(jnp.float32),
        jnp.zeros((13,), jnp.float32),
    ])
    idx = _compute_indices(x_t, y_t, scal, xmap, ymap).reshape(-1)
    part = _das_sc(sinogram.reshape(-1), idx)
    out = part[:NPIX_PAD] + part[NPIX_PAD:]
    return out[:NPIX].reshape(GRID, GRID)


# speedup vs baseline: 1290.0403x; 1.1784x over previous
import functools

import jax
import jax.numpy as jnp
from jax import lax
from jax.experimental import pallas as pl
from jax.experimental.pallas import tpu as pltpu
from jax.experimental.pallas import tpu_sc as plsc

R_RING = 0.05
N_TR = 512
T_SAMPLE = 3.75e-05
N_TIME = 2048
GRID = 200
NPIX = GRID * GRID
NPIX_PAD = 40960
BROWS = NPIX_PAD // 128
NC = 2
NS = 16
NW = NC * NS
T_PER_W = N_TR // NW
HPIX = NPIX_PAD // 2
HROWS = BROWS // 2
CHUNK = 4096
NCHUNK = HPIX // CHUNK
GPC = CHUNK // 16
UNROLL = 4
STRIPE = NPIX_PAD // NS
SGROUPS = STRIPE // 16

_MAGIC = 8388608.0


T_BLK = 8


def _idx_tc_kernel(xt_ref, yt_ref, scal_ref, xmap_ref, ymap_ref, out_ref):
    pid = pl.program_id(0)
    v0 = scal_ref[0]
    dd = scal_ref[1]
    re = scal_ref[2]
    vts = v0 * jnp.float32(T_SAMPLE)
    x = xmap_ref[...]
    y = ymap_ref[...]

    def _idx_half(xt, yt, xh, yh):
        dx = xt - xh
        dy = yt - yh
        dist = jnp.sqrt(dx * dx + dy * dy)
        q = ((dist + re) - dd) / vts
        rr = (q + _MAGIC) - _MAGIC
        rr = jnp.minimum(jnp.maximum(rr, jnp.float32(0.0)),
                         jnp.float32(N_TIME - 1))
        return rr.astype(jnp.int32)

    for r in range(T_BLK):
        t = pid * T_BLK + r
        lo = _idx_half(xt_ref[t], yt_ref[t], x[:HROWS], y[:HROWS])
        hi = _idx_half(xt_ref[t], yt_ref[t], x[HROWS:], y[HROWS:])
        out_ref[r] = lo | (hi << 16)


def _compute_indices(xt, yt, scal, xmap, ymap):
    return pl.pallas_call(
        _idx_tc_kernel,
        grid=(N_TR // T_BLK,),
        in_specs=[
            pl.BlockSpec(memory_space=pltpu.SMEM),
            pl.BlockSpec(memory_space=pltpu.SMEM),
            pl.BlockSpec(memory_space=pltpu.SMEM),
            pl.BlockSpec((BROWS, 128), lambda g: (0, 0)),
            pl.BlockSpec((BROWS, 128), lambda g: (0, 0)),
        ],
        out_specs=pl.BlockSpec((T_BLK, HROWS, 128), lambda g: (g, 0, 0)),
        out_shape=jax.ShapeDtypeStruct((N_TR, HROWS, 128), jnp.int32),
    )(xt, yt, scal, xmap, ymap)


_mesh = plsc.VectorSubcoreMesh(core_axis_name="c", subcore_axis_name="s")


@functools.partial(
    pl.kernel,
    out_type=jax.ShapeDtypeStruct((NC * NPIX_PAD,), jnp.float32),
    mesh=_mesh,
    scratch_types=[
        pltpu.VMEM((NPIX_PAD,), jnp.float32),
        pltpu.VMEM((N_TIME,), jnp.float32),
        pltpu.VMEM((CHUNK,), jnp.int32),
        pltpu.VMEM((CHUNK,), jnp.int32),
        pltpu.VMEM((STRIPE,), jnp.float32),
        pltpu.VMEM((STRIPE,), jnp.float32),
        pltpu.VMEM_SHARED((NS * NPIX_PAD,), jnp.float32),
        pltpu.SemaphoreType.DMA,
        pltpu.SemaphoreType.DMA,
    ],
    compiler_params=pltpu.CompilerParams(needs_layout_passes=False),
)
def _das_sc(sino_hbm, idx_hbm, out_hbm,
            acc_v, rows_v, idx_a, idx_b, sin_v, sacc_v, shared,
            sem_a, sem_b):
    c = lax.axis_index("c")
    s = lax.axis_index("s")
    w = c * NS + s

    lane = lax.iota(jnp.int32, 16)
    head_mask = jnp.where(lane == 0, jnp.float32(0), jnp.float32(1))
    tail_mask = jnp.where(lane == 15, jnp.float32(0), jnp.float32(1))
    zero16 = jnp.zeros((16,), jnp.float32)

    def zacc(i, carry):
        base = i * 128
        for u in range(8):
            acc_v[pl.ds(base + u * 16, 16)] = zero16
        return carry

    lax.fori_loop(0, NPIX_PAD // 128, zacc, 0)

    t0 = w * T_PER_W

    def _gather_chunk(idx_v, row_ref, base):
        @plsc.parallel_loop(0, GPC, step=1, unroll=UNROLL)
        def _(g):
            off = g * 16
            w16 = idx_v[pl.ds(off, 16)]
            ilo = w16 & jnp.int32(0xFFFF)
            ihi = lax.shift_right_logical(w16, jnp.int32(16))
            vlo = plsc.load_gather(row_ref, [ilo])
            vhi = plsc.load_gather(row_ref, [ihi])
            aoff = base + off
            plsc.addupdate(acc_v.at[pl.ds(aoff, 16)], vlo)
            plsc.addupdate(acc_v.at[pl.ds(HPIX + aoff, 16)], vhi)

    def t_body(jt, carry):
        t = t0 + jt
        pltpu.sync_copy(sino_hbm.at[pl.ds(t * N_TIME, N_TIME)], rows_v)
        rows_v[pl.ds(0, 16)] = rows_v[pl.ds(0, 16)] * head_mask
        rows_v[pl.ds(N_TIME - 16, 16)] = (
            rows_v[pl.ds(N_TIME - 16, 16)] * tail_mask)

        bufs = (idx_a, idx_b)
        sems = (sem_a, sem_b)
        dbase = t * HPIX
        pend = [pltpu.async_copy(idx_hbm.at[pl.ds(dbase, CHUNK)],
                                 bufs[0], sems[0])]

        for cc in range(NCHUNK):
            if cc + 1 < NCHUNK:
                nxt = (cc + 1) % 2
                pend.append(pltpu.async_copy(
                    idx_hbm.at[pl.ds(dbase + (cc + 1) * CHUNK, CHUNK)],
                    bufs[nxt], sems[nxt]))
            pend[cc].wait()
            _gather_chunk(bufs[cc % 2], rows_v, cc * CHUNK)
        return carry

    lax.fori_loop(0, T_PER_W, t_body, 0)

    pltpu.sync_copy(acc_v, shared.at[pl.ds(s * NPIX_PAD, NPIX_PAD)])
    plsc.subcore_barrier()

    sbase = s * STRIPE
    pltpu.sync_copy(shared.at[pl.ds(sbase, STRIPE)], sacc_v)

    def r_body(t2, carry):
        pltpu.sync_copy(shared.at[pl.ds(t2 * NPIX_PAD + sbase, STRIPE)], sin_v)

        def a_body(g, carry2):
            base = g * 128
            for u in range(8):
                off = base + u * 16
                sacc_v[pl.ds(off, 16)] = (sacc_v[pl.ds(off, 16)]
                                          + sin_v[pl.ds(off, 16)])
            return carry2

        lax.fori_loop(0, SGROUPS // 8, a_body, 0)
        return carry

    lax.fori_loop(1, NS, r_body, 0)

    scale = jnp.full((16,), jnp.float32(1.0 / N_TR), jnp.float32)

    def s_body(g, carry):
        base = g * 128
        for u in range(8):
            off = base + u * 16
            sacc_v[pl.ds(off, 16)] = sacc_v[pl.ds(off, 16)] * scale
        return carry

    lax.fori_loop(0, SGROUPS // 8, s_body, 0)
    pltpu.sync_copy(sacc_v, out_hbm.at[pl.ds(c * NPIX_PAD + sbase, STRIPE)])


def kernel(sinogram, v0, d_delay, ring_error):
    angle = (2.0 * jnp.pi / N_TR) * (jnp.arange(N_TR, dtype=jnp.float32) + 1.0)
    x_t = R_RING * jnp.cos(angle - jnp.pi)
    y_t = R_RING * jnp.sin(angle - jnp.pi)
    p = jnp.arange(NPIX_PAD, dtype=jnp.int32)
    xmap = (jnp.float32(-0.02)
            + jnp.float32(0.0002) * (p // GRID).astype(jnp.float32))
    ymap = (jnp.float32(-0.02)
            + jnp.float32(0.0002) * (p % GRID).astype(jnp.float32))
    xmap = xmap.reshape(BROWS, 128)
    ymap = ymap.reshape(BROWS, 128)
    scal = jnp.concatenate([
        v0.astype(jnp.float32),
        d_delay.astype(jnp.float32),
        ring_error.astype(jnp.float32),
        jnp.zeros((13,), jnp.float32),
    ])
    idx = _compute_indices(x_t, y_t, scal, xmap, ymap).reshape(-1)
    part = _das_sc(sinogram.reshape(-1), idx)
    out = part[:NPIX_PAD] + part[NPIX_PAD:]
    return out[:NPIX].reshape(GRID, GRID)


# speedup vs baseline: 1399.9694x; 1.0852x over previous
import functools

import jax
import jax.numpy as jnp
from jax import lax
from jax.experimental import pallas as pl
from jax.experimental.pallas import tpu as pltpu
from jax.experimental.pallas import tpu_sc as plsc

R_RING = 0.05
N_TR = 512
T_SAMPLE = 3.75e-05
N_TIME = 2048
GRID = 200
NPIX = GRID * GRID
NPIX_PAD = 40960
BROWS = NPIX_PAD // 128
NC = 2
NS = 16
NW = NC * NS
T_PER_W = N_TR // NW
HPIX = NPIX_PAD // 2
HROWS = BROWS // 2
CHUNK = 4096
NCHUNK = HPIX // CHUNK
GPC = CHUNK // 16
UNROLL = 8
STRIPE = NPIX_PAD // NS
SGROUPS = STRIPE // 16

_MAGIC = 8388608.0


T_BLK = 16


def _idx_tc_kernel(xt_ref, yt_ref, scal_ref, xmap_ref, ymap_ref, out_ref):
    pid = pl.program_id(0)
    v0 = scal_ref[0]
    dd = scal_ref[1]
    re = scal_ref[2]
    vts = v0 * jnp.float32(T_SAMPLE)
    x = xmap_ref[...]
    y = ymap_ref[...]

    def _idx_half(xt, yt, xh, yh):
        dx = xt - xh
        dy = yt - yh
        dist = jnp.sqrt(dx * dx + dy * dy)
        q = ((dist + re) - dd) / vts
        rr = (q + _MAGIC) - _MAGIC
        rr = jnp.minimum(jnp.maximum(rr, jnp.float32(0.0)),
                         jnp.float32(N_TIME - 1))
        return rr.astype(jnp.int32)

    for r in range(T_BLK):
        t = pid * T_BLK + r
        lo = _idx_half(xt_ref[t], yt_ref[t], x[:HROWS], y[:HROWS])
        hi = _idx_half(xt_ref[t], yt_ref[t], x[HROWS:], y[HROWS:])
        out_ref[r] = lo | (hi << 16)


def _compute_indices(xt, yt, scal, xmap, ymap):
    return pl.pallas_call(
        _idx_tc_kernel,
        grid=(N_TR // T_BLK,),
        in_specs=[
            pl.BlockSpec(memory_space=pltpu.SMEM),
            pl.BlockSpec(memory_space=pltpu.SMEM),
            pl.BlockSpec(memory_space=pltpu.SMEM),
            pl.BlockSpec((BROWS, 128), lambda g: (0, 0)),
            pl.BlockSpec((BROWS, 128), lambda g: (0, 0)),
        ],
        out_specs=pl.BlockSpec((T_BLK, HROWS, 128), lambda g: (g, 0, 0)),
        out_shape=jax.ShapeDtypeStruct((N_TR, HROWS, 128), jnp.int32),
    )(xt, yt, scal, xmap, ymap)


_mesh = plsc.VectorSubcoreMesh(core_axis_name="c", subcore_axis_name="s")


@functools.partial(
    pl.kernel,
    out_type=jax.ShapeDtypeStruct((NC * NPIX_PAD,), jnp.float32),
    mesh=_mesh,
    scratch_types=[
        pltpu.VMEM((NPIX_PAD,), jnp.float32),
        pltpu.VMEM((N_TIME,), jnp.float32),
        pltpu.VMEM((CHUNK,), jnp.int32),
        pltpu.VMEM((CHUNK,), jnp.int32),
        pltpu.VMEM((STRIPE,), jnp.float32),
        pltpu.VMEM((STRIPE,), jnp.float32),
        pltpu.VMEM_SHARED((NS * NPIX_PAD,), jnp.float32),
        pltpu.SemaphoreType.DMA,
        pltpu.SemaphoreType.DMA,
    ],
    compiler_params=pltpu.CompilerParams(needs_layout_passes=False),
)
def _das_sc(sino_hbm, idx_hbm, out_hbm,
            acc_v, rows_v, idx_a, idx_b, sin_v, sacc_v, shared,
            sem_a, sem_b):
    c = lax.axis_index("c")
    s = lax.axis_index("s")
    w = c * NS + s

    lane = lax.iota(jnp.int32, 16)
    head_mask = jnp.where(lane == 0, jnp.float32(0), jnp.float32(1))
    tail_mask = jnp.where(lane == 15, jnp.float32(0), jnp.float32(1))
    zero16 = jnp.zeros((16,), jnp.float32)

    def zacc(i, carry):
        base = i * 128
        for u in range(8):
            acc_v[pl.ds(base + u * 16, 16)] = zero16
        return carry

    lax.fori_loop(0, NPIX_PAD // 128, zacc, 0)

    t0 = w * T_PER_W

    def _gather_chunk(idx_v, row_ref, base):
        @plsc.parallel_loop(0, GPC, step=1, unroll=UNROLL)
        def _(g):
            off = g * 16
            w16 = idx_v[pl.ds(off, 16)]
            ilo = w16 & jnp.int32(0xFFFF)
            ihi = lax.shift_right_logical(w16, jnp.int32(16))
            vlo = plsc.load_gather(row_ref, [ilo])
            vhi = plsc.load_gather(row_ref, [ihi])
            aoff = base + off
            plsc.addupdate(acc_v.at[pl.ds(aoff, 16)], vlo)
            plsc.addupdate(acc_v.at[pl.ds(HPIX + aoff, 16)], vhi)

    def t_body(jt, carry):
        t = t0 + jt
        pltpu.sync_copy(sino_hbm.at[pl.ds(t * N_TIME, N_TIME)], rows_v)
        rows_v[pl.ds(0, 16)] = rows_v[pl.ds(0, 16)] * head_mask
        rows_v[pl.ds(N_TIME - 16, 16)] = (
            rows_v[pl.ds(N_TIME - 16, 16)] * tail_mask)

        bufs = (idx_a, idx_b)
        sems = (sem_a, sem_b)
        dbase = t * HPIX
        pend = [pltpu.async_copy(idx_hbm.at[pl.ds(dbase, CHUNK)],
                                 bufs[0], sems[0])]

        for cc in range(NCHUNK):
            if cc + 1 < NCHUNK:
                nxt = (cc + 1) % 2
                pend.append(pltpu.async_copy(
                    idx_hbm.at[pl.ds(dbase + (cc + 1) * CHUNK, CHUNK)],
                    bufs[nxt], sems[nxt]))
            pend[cc].wait()
            _gather_chunk(bufs[cc % 2], rows_v, cc * CHUNK)
        return carry

    lax.fori_loop(0, T_PER_W, t_body, 0)

    pltpu.sync_copy(acc_v, shared.at[pl.ds(s * NPIX_PAD, NPIX_PAD)])
    plsc.subcore_barrier()

    sbase = s * STRIPE
    pltpu.sync_copy(shared.at[pl.ds(sbase, STRIPE)], sacc_v)

    def r_body(t2, carry):
        pltpu.sync_copy(shared.at[pl.ds(t2 * NPIX_PAD + sbase, STRIPE)], sin_v)

        def a_body(g, carry2):
            base = g * 128
            for u in range(8):
                off = base + u * 16
                sacc_v[pl.ds(off, 16)] = (sacc_v[pl.ds(off, 16)]
                                          + sin_v[pl.ds(off, 16)])
            return carry2

        lax.fori_loop(0, SGROUPS // 8, a_body, 0)
        return carry

    lax.fori_loop(1, NS, r_body, 0)

    scale = jnp.full((16,), jnp.float32(1.0 / N_TR), jnp.float32)

    def s_body(g, carry):
        base = g * 128
        for u in range(8):
            off = base + u * 16
            sacc_v[pl.ds(off, 16)] = sacc_v[pl.ds(off, 16)] * scale
        return carry

    lax.fori_loop(0, SGROUPS // 8, s_body, 0)
    pltpu.sync_copy(sacc_v, out_hbm.at[pl.ds(c * NPIX_PAD + sbase, STRIPE)])


def kernel(sinogram, v0, d_delay, ring_error):
    angle = (2.0 * jnp.pi / N_TR) * (jnp.arange(N_TR, dtype=jnp.float32) + 1.0)
    x_t = R_RING * jnp.cos(angle - jnp.pi)
    y_t = R_RING * jnp.sin(angle - jnp.pi)
    p = jnp.arange(NPIX_PAD, dtype=jnp.int32)
    xmap = (jnp.float32(-0.02)
            + jnp.float32(0.0002) * (p // GRID).astype(jnp.float32))
    ymap = (jnp.float32(-0.02)
            + jnp.float32(0.0002) * (p % GRID).astype(jnp.float32))
    xmap = xmap.reshape(BROWS, 128)
    ymap = ymap.reshape(BROWS, 128)
    scal = jnp.concatenate([
        v0.astype(jnp.float32),
        d_delay.astype(jnp.float32),
        ring_error.astype(jnp.float32),
        jnp.zeros((13,), jnp.float32),
    ])
    idx = _compute_indices(x_t, y_t, scal, xmap, ymap).reshape(-1)
    part = _das_sc(sinogram.reshape(-1), idx)
    out = part[:NPIX_PAD] + part[NPIX_PAD:]
    return out[:NPIX].reshape(GRID, GRID)


# speedup vs baseline: 1413.4769x; 1.0096x over previous
import functools

import jax
import jax.numpy as jnp
from jax import lax
from jax.experimental import pallas as pl
from jax.experimental.pallas import tpu as pltpu
from jax.experimental.pallas import tpu_sc as plsc

R_RING = 0.05
N_TR = 512
T_SAMPLE = 3.75e-05
N_TIME = 2048
GRID = 200
NPIX = GRID * GRID
NPIX_PAD = 40960
BROWS = NPIX_PAD // 128
N_STAGE = 2
N_TR_S = N_TR // N_STAGE
NC = 2
NS = 16
NW = NC * NS
T_PER_W = N_TR_S // NW
HPIX = NPIX_PAD // 2
HROWS = BROWS // 2
CHUNK = 4096
NCHUNK = HPIX // CHUNK
GPC = CHUNK // 16
UNROLL = 8
STRIPE = NPIX_PAD // NS
SGROUPS = STRIPE // 16

_MAGIC = 8388608.0


T_BLK = 16


def _idx_tc_kernel(xt_ref, yt_ref, scal_ref, xmap_ref, ymap_ref, out_ref):
    pid = pl.program_id(0)
    v0 = scal_ref[0]
    dd = scal_ref[1]
    re = scal_ref[2]
    vts = v0 * jnp.float32(T_SAMPLE)
    x = xmap_ref[...]
    y = ymap_ref[...]

    def _idx_half(xt, yt, xh, yh):
        dx = xt - xh
        dy = yt - yh
        dist = jnp.sqrt(dx * dx + dy * dy)
        q = ((dist + re) - dd) / vts
        rr = (q + _MAGIC) - _MAGIC
        rr = jnp.minimum(jnp.maximum(rr, jnp.float32(0.0)),
                         jnp.float32(N_TIME - 1))
        return rr.astype(jnp.int32)

    for r in range(T_BLK):
        t = pid * T_BLK + r
        lo = _idx_half(xt_ref[t], yt_ref[t], x[:HROWS], y[:HROWS])
        hi = _idx_half(xt_ref[t], yt_ref[t], x[HROWS:], y[HROWS:])
        out_ref[r] = lo | (hi << 16)


def _compute_indices(xt, yt, scal, xmap, ymap):
    return pl.pallas_call(
        _idx_tc_kernel,
        grid=(N_TR_S // T_BLK,),
        in_specs=[
            pl.BlockSpec(memory_space=pltpu.SMEM),
            pl.BlockSpec(memory_space=pltpu.SMEM),
            pl.BlockSpec(memory_space=pltpu.SMEM),
            pl.BlockSpec((BROWS, 128), lambda g: (0, 0)),
            pl.BlockSpec((BROWS, 128), lambda g: (0, 0)),
        ],
        out_specs=pl.BlockSpec((T_BLK, HROWS, 128), lambda g: (g, 0, 0)),
        out_shape=jax.ShapeDtypeStruct((N_TR_S, HROWS, 128), jnp.int32),
    )(xt, yt, scal, xmap, ymap)


_mesh = plsc.VectorSubcoreMesh(core_axis_name="c", subcore_axis_name="s")


@functools.partial(
    pl.kernel,
    out_type=jax.ShapeDtypeStruct((NC * NPIX_PAD,), jnp.float32),
    mesh=_mesh,
    scratch_types=[
        pltpu.VMEM((NPIX_PAD,), jnp.float32),
        pltpu.VMEM((N_TIME,), jnp.float32),
        pltpu.VMEM((CHUNK,), jnp.int32),
        pltpu.VMEM((CHUNK,), jnp.int32),
        pltpu.VMEM((STRIPE,), jnp.float32),
        pltpu.VMEM((STRIPE,), jnp.float32),
        pltpu.VMEM_SHARED((NS * NPIX_PAD,), jnp.float32),
        pltpu.SemaphoreType.DMA,
        pltpu.SemaphoreType.DMA,
    ],
    compiler_params=pltpu.CompilerParams(needs_layout_passes=False),
)
def _das_sc(sino_hbm, idx_hbm, out_hbm,
            acc_v, rows_v, idx_a, idx_b, sin_v, sacc_v, shared,
            sem_a, sem_b):
    c = lax.axis_index("c")
    s = lax.axis_index("s")
    w = c * NS + s

    lane = lax.iota(jnp.int32, 16)
    head_mask = jnp.where(lane == 0, jnp.float32(0), jnp.float32(1))
    tail_mask = jnp.where(lane == 15, jnp.float32(0), jnp.float32(1))
    zero16 = jnp.zeros((16,), jnp.float32)

    def zacc(i, carry):
        base = i * 128
        for u in range(8):
            acc_v[pl.ds(base + u * 16, 16)] = zero16
        return carry

    lax.fori_loop(0, NPIX_PAD // 128, zacc, 0)

    t0 = w * T_PER_W

    def _gather_chunk(idx_v, row_ref, base):
        @plsc.parallel_loop(0, GPC, step=1, unroll=UNROLL)
        def _(g):
            off = g * 16
            w16 = idx_v[pl.ds(off, 16)]
            ilo = w16 & jnp.int32(0xFFFF)
            ihi = lax.shift_right_logical(w16, jnp.int32(16))
            vlo = plsc.load_gather(row_ref, [ilo])
            vhi = plsc.load_gather(row_ref, [ihi])
            aoff = base + off
            plsc.addupdate(acc_v.at[pl.ds(aoff, 16)], vlo)
            plsc.addupdate(acc_v.at[pl.ds(HPIX + aoff, 16)], vhi)

    def t_body(jt, carry):
        t = t0 + jt
        pltpu.sync_copy(sino_hbm.at[pl.ds(t * N_TIME, N_TIME)], rows_v)
        rows_v[pl.ds(0, 16)] = rows_v[pl.ds(0, 16)] * head_mask
        rows_v[pl.ds(N_TIME - 16, 16)] = (
            rows_v[pl.ds(N_TIME - 16, 16)] * tail_mask)

        bufs = (idx_a, idx_b)
        sems = (sem_a, sem_b)
        dbase = t * HPIX
        pend = [pltpu.async_copy(idx_hbm.at[pl.ds(dbase, CHUNK)],
                                 bufs[0], sems[0])]

        for cc in range(NCHUNK):
            if cc + 1 < NCHUNK:
                nxt = (cc + 1) % 2
                pend.append(pltpu.async_copy(
                    idx_hbm.at[pl.ds(dbase + (cc + 1) * CHUNK, CHUNK)],
                    bufs[nxt], sems[nxt]))
            pend[cc].wait()
            _gather_chunk(bufs[cc % 2], rows_v, cc * CHUNK)
        return carry

    lax.fori_loop(0, T_PER_W, t_body, 0)

    pltpu.sync_copy(acc_v, shared.at[pl.ds(s * NPIX_PAD, NPIX_PAD)])
    plsc.subcore_barrier()

    sbase = s * STRIPE
    pltpu.sync_copy(shared.at[pl.ds(sbase, STRIPE)], sacc_v)

    def r_body(t2, carry):
        pltpu.sync_copy(shared.at[pl.ds(t2 * NPIX_PAD + sbase, STRIPE)], sin_v)

        def a_body(g, carry2):
            base = g * 128
            for u in range(8):
                off = base + u * 16
                sacc_v[pl.ds(off, 16)] = (sacc_v[pl.ds(off, 16)]
                                          + sin_v[pl.ds(off, 16)])
            return carry2

        lax.fori_loop(0, SGROUPS // 8, a_body, 0)
        return carry

    lax.fori_loop(1, NS, r_body, 0)

    scale = jnp.full((16,), jnp.float32(1.0 / N_TR), jnp.float32)

    def s_body(g, carry):
        base = g * 128
        for u in range(8):
            off = base + u * 16
            sacc_v[pl.ds(off, 16)] = sacc_v[pl.ds(off, 16)] * scale
        return carry

    lax.fori_loop(0, SGROUPS // 8, s_body, 0)
    pltpu.sync_copy(sacc_v, out_hbm.at[pl.ds(c * NPIX_PAD + sbase, STRIPE)])


def kernel(sinogram, v0, d_delay, ring_error):
    angle = (2.0 * jnp.pi / N_TR) * (jnp.arange(N_TR, dtype=jnp.float32) + 1.0)
    x_t = R_RING * jnp.cos(angle - jnp.pi)
    y_t = R_RING * jnp.sin(angle - jnp.pi)
    p = jnp.arange(NPIX_PAD, dtype=jnp.int32)
    xmap = (jnp.float32(-0.02)
            + jnp.float32(0.0002) * (p // GRID).astype(jnp.float32))
    ymap = (jnp.float32(-0.02)
            + jnp.float32(0.0002) * (p % GRID).astype(jnp.float32))
    xmap = xmap.reshape(BROWS, 128)
    ymap = ymap.reshape(BROWS, 128)
    scal = jnp.concatenate([
        v0.astype(jnp.float32),
        d_delay.astype(jnp.float32),
        ring_error.astype(jnp.float32),
        jnp.zeros((13,), jnp.float32),
    ])
    sino_flat = sinogram.reshape(-1)
    out = None
    for h in range(N_STAGE):
        xt_h = lax.slice(x_t, (h * N_TR_S,), ((h + 1) * N_TR_S,))
        yt_h = lax.slice(y_t, (h * N_TR_S,), ((h + 1) * N_TR_S,))
        idx_h = _compute_indices(xt_h, yt_h, scal, xmap, ymap).reshape(-1)
        sino_h = lax.slice(sino_flat, (h * N_TR_S * N_TIME,),
                           ((h + 1) * N_TR_S * N_TIME,))
        part = _das_sc(sino_h, idx_h)
        stage = part[:NPIX_PAD] + part[NPIX_PAD:]
        out = stage if out is None else out + stage
    return out[:NPIX].reshape(GRID, GRID)


# speedup vs baseline: 1490.9007x; 1.0548x over previous
import functools

import jax
import jax.numpy as jnp
from jax import lax
from jax.experimental import pallas as pl
from jax.experimental.pallas import tpu as pltpu
from jax.experimental.pallas import tpu_sc as plsc

R_RING = 0.05
N_TR = 512
T_SAMPLE = 3.75e-05
N_TIME = 2048
GRID = 200
NPIX = GRID * GRID
NPIX_PAD = 40960
BROWS = NPIX_PAD // 128
N_STAGE = 2
N_TR_S = N_TR // N_STAGE
NC = 2
NS = 16
NW = NC * NS
T_PER_W = N_TR_S // NW
HPIX = NPIX_PAD // 2
HROWS = BROWS // 2
CHUNK = 4096
NCHUNK = HPIX // CHUNK
GPC = CHUNK // 16
UNROLL = 8
STRIPE = NPIX_PAD // NS
SGROUPS = STRIPE // 16

_MAGIC = 8388608.0


T_BLK = 16


def _idx_tc_kernel(xt_ref, yt_ref, scal_ref, xmap_ref, ymap_ref, out_ref):
    pid = pl.program_id(0)
    v0 = scal_ref[0]
    dd = scal_ref[1]
    re = scal_ref[2]
    vts = v0 * jnp.float32(T_SAMPLE)
    x = xmap_ref[...]
    y = ymap_ref[...]

    def _idx_half(xt, yt, xh, yh):
        dx = xt - xh
        dy = yt - yh
        dist = jnp.sqrt(dx * dx + dy * dy)
        q = ((dist + re) - dd) / vts
        rr = (q + _MAGIC) - _MAGIC
        rr = jnp.minimum(jnp.maximum(rr, jnp.float32(0.0)),
                         jnp.float32(N_TIME - 1))
        return rr.astype(jnp.int32)

    for r in range(T_BLK):
        t = pid * T_BLK + r
        lo = _idx_half(xt_ref[t], yt_ref[t], x[:HROWS], y[:HROWS])
        hi = _idx_half(xt_ref[t], yt_ref[t], x[HROWS:], y[HROWS:])
        out_ref[r] = lo | (hi << 16)


def _compute_indices(xt, yt, scal, xmap, ymap):
    return pl.pallas_call(
        _idx_tc_kernel,
        grid=(N_TR_S // T_BLK,),
        in_specs=[
            pl.BlockSpec(memory_space=pltpu.SMEM),
            pl.BlockSpec(memory_space=pltpu.SMEM),
            pl.BlockSpec(memory_space=pltpu.SMEM),
            pl.BlockSpec((BROWS, 128), lambda g: (0, 0)),
            pl.BlockSpec((BROWS, 128), lambda g: (0, 0)),
        ],
        out_specs=pl.BlockSpec((T_BLK, HROWS, 128), lambda g: (g, 0, 0)),
        out_shape=jax.ShapeDtypeStruct((N_TR_S, HROWS, 128), jnp.int32),
    )(xt, yt, scal, xmap, ymap)


_mesh = plsc.VectorSubcoreMesh(core_axis_name="c", subcore_axis_name="s")


@functools.partial(
    pl.kernel,
    out_type=jax.ShapeDtypeStruct((NC * NPIX_PAD,), jnp.float32),
    mesh=_mesh,
    scratch_types=[
        pltpu.VMEM((NPIX_PAD,), jnp.float32),
        pltpu.VMEM((N_TIME,), jnp.float32),
        pltpu.VMEM((N_TIME,), jnp.float32),
        pltpu.VMEM((CHUNK,), jnp.int32),
        pltpu.VMEM((CHUNK,), jnp.int32),
        pltpu.VMEM((STRIPE,), jnp.float32),
        pltpu.VMEM((STRIPE,), jnp.float32),
        pltpu.VMEM_SHARED((NS * NPIX_PAD,), jnp.float32),
        pltpu.SemaphoreType.DMA,
        pltpu.SemaphoreType.DMA,
        pltpu.SemaphoreType.DMA,
        pltpu.SemaphoreType.DMA,
    ],
    compiler_params=pltpu.CompilerParams(needs_layout_passes=False),
)
def _das_sc(sino_hbm, idx_hbm, out_hbm,
            acc_v, row_a, row_b, idx_a, idx_b, sin_v, sacc_v, shared,
            sem_a, sem_b, sem_ra, sem_rb):
    c = lax.axis_index("c")
    s = lax.axis_index("s")
    w = c * NS + s

    lane = lax.iota(jnp.int32, 16)
    head_mask = jnp.where(lane == 0, jnp.float32(0), jnp.float32(1))
    tail_mask = jnp.where(lane == 15, jnp.float32(0), jnp.float32(1))
    zero16 = jnp.zeros((16,), jnp.float32)

    def zacc(i, carry):
        base = i * 128
        for u in range(8):
            acc_v[pl.ds(base + u * 16, 16)] = zero16
        return carry

    lax.fori_loop(0, NPIX_PAD // 128, zacc, 0)

    t0 = w * T_PER_W

    def _gather_chunk(idx_v, row_ref, base):
        @plsc.parallel_loop(0, GPC, step=1, unroll=UNROLL)
        def _(g):
            off = g * 16
            w16 = idx_v[pl.ds(off, 16)]
            ilo = w16 & jnp.int32(0xFFFF)
            ihi = lax.shift_right_logical(w16, jnp.int32(16))
            vlo = plsc.load_gather(row_ref, [ilo])
            vhi = plsc.load_gather(row_ref, [ihi])
            aoff = base + off
            plsc.addupdate(acc_v.at[pl.ds(aoff, 16)], vlo)
            plsc.addupdate(acc_v.at[pl.ds(HPIX + aoff, 16)], vhi)

    ibufs = (idx_a, idx_b)
    isems = (sem_a, sem_b)
    rbufs = (row_a, row_b)
    rsems = (sem_ra, sem_rb)
    slots = [(jt, cc) for jt in range(T_PER_W) for cc in range(NCHUNK)]

    def _issue_idx(k):
        jt, cc = slots[k]
        return pltpu.async_copy(
            idx_hbm.at[pl.ds((t0 + jt) * HPIX + cc * CHUNK, CHUNK)],
            ibufs[k % 2], isems[k % 2])

    def _issue_row(jt):
        return pltpu.async_copy(
            sino_hbm.at[pl.ds((t0 + jt) * N_TIME, N_TIME)],
            rbufs[jt % 2], rsems[jt % 2])

    ipend = {0: _issue_idx(0)}
    rpend = {0: _issue_row(0)}
    for k, (jt, cc) in enumerate(slots):
        if k + 1 < len(slots):
            ipend[k + 1] = _issue_idx(k + 1)
        if cc == 0:
            if jt + 1 < T_PER_W:
                rpend[jt + 1] = _issue_row(jt + 1)
            rpend[jt].wait()
            row = rbufs[jt % 2]
            row[pl.ds(0, 16)] = row[pl.ds(0, 16)] * head_mask
            row[pl.ds(N_TIME - 16, 16)] = (
                row[pl.ds(N_TIME - 16, 16)] * tail_mask)
        ipend[k].wait()
        _gather_chunk(ibufs[k % 2], rbufs[jt % 2], cc * CHUNK)

    pltpu.sync_copy(acc_v, shared.at[pl.ds(s * NPIX_PAD, NPIX_PAD)])
    plsc.subcore_barrier()

    sbase = s * STRIPE
    pltpu.sync_copy(shared.at[pl.ds(sbase, STRIPE)], sacc_v)

    def r_body(t2, carry):
        pltpu.sync_copy(shared.at[pl.ds(t2 * NPIX_PAD + sbase, STRIPE)], sin_v)

        def a_body(g, carry2):
            base = g * 128
            for u in range(8):
                off = base + u * 16
                sacc_v[pl.ds(off, 16)] = (sacc_v[pl.ds(off, 16)]
                                          + sin_v[pl.ds(off, 16)])
            return carry2

        lax.fori_loop(0, SGROUPS // 8, a_body, 0)
        return carry

    lax.fori_loop(1, NS, r_body, 0)

    scale = jnp.full((16,), jnp.float32(1.0 / N_TR), jnp.float32)

    def s_body(g, carry):
        base = g * 128
        for u in range(8):
            off = base + u * 16
            sacc_v[pl.ds(off, 16)] = sacc_v[pl.ds(off, 16)] * scale
        return carry

    lax.fori_loop(0, SGROUPS // 8, s_body, 0)
    pltpu.sync_copy(sacc_v, out_hbm.at[pl.ds(c * NPIX_PAD + sbase, STRIPE)])


def kernel(sinogram, v0, d_delay, ring_error):
    angle = (2.0 * jnp.pi / N_TR) * (jnp.arange(N_TR, dtype=jnp.float32) + 1.0)
    x_t = R_RING * jnp.cos(angle - jnp.pi)
    y_t = R_RING * jnp.sin(angle - jnp.pi)
    p = jnp.arange(NPIX_PAD, dtype=jnp.int32)
    xmap = (jnp.float32(-0.02)
            + jnp.float32(0.0002) * (p // GRID).astype(jnp.float32))
    ymap = (jnp.float32(-0.02)
            + jnp.float32(0.0002) * (p % GRID).astype(jnp.float32))
    xmap = xmap.reshape(BROWS, 128)
    ymap = ymap.reshape(BROWS, 128)
    scal = jnp.concatenate([
        v0.astype(jnp.float32),
        d_delay.astype(jnp.float32),
        ring_error.astype(jnp.float32),
        jnp.zeros((13,), jnp.float32),
    ])
    sino_flat = sinogram.reshape(-1)
    out = None
    for h in range(N_STAGE):
        xt_h = lax.slice(x_t, (h * N_TR_S,), ((h + 1) * N_TR_S,))
        yt_h = lax.slice(y_t, (h * N_TR_S,), ((h + 1) * N_TR_S,))
        idx_h = _compute_indices(xt_h, yt_h, scal, xmap, ymap).reshape(-1)
        sino_h = lax.slice(sino_flat, (h * N_TR_S * N_TIME,),
                           ((h + 1) * N_TR_S * N_TIME,))
        part = _das_sc(sino_h, idx_h)
        stage = part[:NPIX_PAD] + part[NPIX_PAD:]
        out = stage if out is None else out + stage
    return out[:NPIX].reshape(GRID, GRID)


# speedup vs baseline: 1502.6442x; 1.0079x over previous
import functools

import jax
import jax.numpy as jnp
from jax import lax
from jax.experimental import pallas as pl
from jax.experimental.pallas import tpu as pltpu
from jax.experimental.pallas import tpu_sc as plsc

R_RING = 0.05
N_TR = 512
T_SAMPLE = 3.75e-05
N_TIME = 2048
GRID = 200
NPIX = GRID * GRID
NPIX_PAD = 40960
BROWS = NPIX_PAD // 128
STAGES = (96, 160, 256)
NC = 2
NS = 16
NW = NC * NS
HPIX = NPIX_PAD // 2
HROWS = BROWS // 2
CHUNK = 4096
NCHUNK = HPIX // CHUNK
GPC = CHUNK // 16
UNROLL = 8
STRIPE = NPIX_PAD // NS
SGROUPS = STRIPE // 16

_MAGIC = 8388608.0


T_BLK = 16


def _idx_tc_kernel(xt_ref, yt_ref, scal_ref, xmap_ref, ymap_ref, out_ref):
    pid = pl.program_id(0)
    v0 = scal_ref[0]
    dd = scal_ref[1]
    re = scal_ref[2]
    vts = v0 * jnp.float32(T_SAMPLE)
    x = xmap_ref[...]
    y = ymap_ref[...]

    def _idx_half(xt, yt, xh, yh):
        dx = xt - xh
        dy = yt - yh
        dist = jnp.sqrt(dx * dx + dy * dy)
        q = ((dist + re) - dd) / vts
        rr = (q + _MAGIC) - _MAGIC
        rr = jnp.minimum(jnp.maximum(rr, jnp.float32(0.0)),
                         jnp.float32(N_TIME - 1))
        return rr.astype(jnp.int32)

    for r in range(T_BLK):
        t = pid * T_BLK + r
        lo = _idx_half(xt_ref[t], yt_ref[t], x[:HROWS], y[:HROWS])
        hi = _idx_half(xt_ref[t], yt_ref[t], x[HROWS:], y[HROWS:])
        out_ref[r] = lo | (hi << 16)


def _compute_indices(xt, yt, scal, xmap, ymap, n_tr_s):
    return pl.pallas_call(
        _idx_tc_kernel,
        grid=(n_tr_s // T_BLK,),
        in_specs=[
            pl.BlockSpec(memory_space=pltpu.SMEM),
            pl.BlockSpec(memory_space=pltpu.SMEM),
            pl.BlockSpec(memory_space=pltpu.SMEM),
            pl.BlockSpec((BROWS, 128), lambda g: (0, 0)),
            pl.BlockSpec((BROWS, 128), lambda g: (0, 0)),
        ],
        out_specs=pl.BlockSpec((T_BLK, HROWS, 128), lambda g: (g, 0, 0)),
        out_shape=jax.ShapeDtypeStruct((n_tr_s, HROWS, 128), jnp.int32),
    )(xt, yt, scal, xmap, ymap)


_mesh = plsc.VectorSubcoreMesh(core_axis_name="c", subcore_axis_name="s")


def _make_das_sc(t_per_w):
    @functools.partial(
        pl.kernel,
        out_type=jax.ShapeDtypeStruct((NC * NPIX_PAD,), jnp.float32),
        mesh=_mesh,
        scratch_types=[
            pltpu.VMEM((NPIX_PAD,), jnp.float32),
            pltpu.VMEM((N_TIME,), jnp.float32),
            pltpu.VMEM((N_TIME,), jnp.float32),
            pltpu.VMEM((CHUNK,), jnp.int32),
            pltpu.VMEM((CHUNK,), jnp.int32),
            pltpu.VMEM((STRIPE,), jnp.float32),
            pltpu.VMEM((STRIPE,), jnp.float32),
            pltpu.VMEM_SHARED((NS * NPIX_PAD,), jnp.float32),
            pltpu.SemaphoreType.DMA,
            pltpu.SemaphoreType.DMA,
            pltpu.SemaphoreType.DMA,
            pltpu.SemaphoreType.DMA,
        ],
        compiler_params=pltpu.CompilerParams(needs_layout_passes=False),
    )
    def _das_sc(sino_hbm, idx_hbm, out_hbm,
                acc_v, row_a, row_b, idx_a, idx_b, sin_v, sacc_v, shared,
                sem_a, sem_b, sem_ra, sem_rb):
        return _das_sc_body(
            t_per_w, sino_hbm, idx_hbm, out_hbm, acc_v, row_a, row_b,
            idx_a, idx_b, sin_v, sacc_v, shared, sem_a, sem_b, sem_ra,
            sem_rb)

    return _das_sc


def _das_sc_body(T_PER_W, sino_hbm, idx_hbm, out_hbm,
                 acc_v, row_a, row_b, idx_a, idx_b, sin_v, sacc_v, shared,
                 sem_a, sem_b, sem_ra, sem_rb):
    c = lax.axis_index("c")
    s = lax.axis_index("s")
    w = c * NS + s

    lane = lax.iota(jnp.int32, 16)
    head_mask = jnp.where(lane == 0, jnp.float32(0), jnp.float32(1))
    tail_mask = jnp.where(lane == 15, jnp.float32(0), jnp.float32(1))
    zero16 = jnp.zeros((16,), jnp.float32)

    def zacc(i, carry):
        base = i * 128
        for u in range(8):
            acc_v[pl.ds(base + u * 16, 16)] = zero16
        return carry

    lax.fori_loop(0, NPIX_PAD // 128, zacc, 0)

    t0 = w * T_PER_W

    def _gather_chunk(idx_v, row_ref, base):
        @plsc.parallel_loop(0, GPC, step=1, unroll=UNROLL)
        def _(g):
            off = g * 16
            w16 = idx_v[pl.ds(off, 16)]
            ilo = w16 & jnp.int32(0xFFFF)
            ihi = lax.shift_right_logical(w16, jnp.int32(16))
            vlo = plsc.load_gather(row_ref, [ilo])
            vhi = plsc.load_gather(row_ref, [ihi])
            aoff = base + off
            plsc.addupdate(acc_v.at[pl.ds(aoff, 16)], vlo)
            plsc.addupdate(acc_v.at[pl.ds(HPIX + aoff, 16)], vhi)

    ibufs = (idx_a, idx_b)
    isems = (sem_a, sem_b)
    rbufs = (row_a, row_b)
    rsems = (sem_ra, sem_rb)
    slots = [(jt, cc) for jt in range(T_PER_W) for cc in range(NCHUNK)]

    def _issue_idx(k):
        jt, cc = slots[k]
        return pltpu.async_copy(
            idx_hbm.at[pl.ds((t0 + jt) * HPIX + cc * CHUNK, CHUNK)],
            ibufs[k % 2], isems[k % 2])

    def _issue_row(jt):
        return pltpu.async_copy(
            sino_hbm.at[pl.ds((t0 + jt) * N_TIME, N_TIME)],
            rbufs[jt % 2], rsems[jt % 2])

    ipend = {0: _issue_idx(0)}
    rpend = {0: _issue_row(0)}
    for k, (jt, cc) in enumerate(slots):
        if k + 1 < len(slots):
            ipend[k + 1] = _issue_idx(k + 1)
        if cc == 0:
            if jt + 1 < T_PER_W:
                rpend[jt + 1] = _issue_row(jt + 1)
            rpend[jt].wait()
            row = rbufs[jt % 2]
            row[pl.ds(0, 16)] = row[pl.ds(0, 16)] * head_mask
            row[pl.ds(N_TIME - 16, 16)] = (
                row[pl.ds(N_TIME - 16, 16)] * tail_mask)
        ipend[k].wait()
        _gather_chunk(ibufs[k % 2], rbufs[jt % 2], cc * CHUNK)

    pltpu.sync_copy(acc_v, shared.at[pl.ds(s * NPIX_PAD, NPIX_PAD)])
    plsc.subcore_barrier()

    sbase = s * STRIPE
    pltpu.sync_copy(shared.at[pl.ds(sbase, STRIPE)], sacc_v)

    def r_body(t2, carry):
        pltpu.sync_copy(shared.at[pl.ds(t2 * NPIX_PAD + sbase, STRIPE)], sin_v)

        def a_body(g, carry2):
            base = g * 128
            for u in range(8):
                off = base + u * 16
                sacc_v[pl.ds(off, 16)] = (sacc_v[pl.ds(off, 16)]
                                          + sin_v[pl.ds(off, 16)])
            return carry2

        lax.fori_loop(0, SGROUPS // 8, a_body, 0)
        return carry

    lax.fori_loop(1, NS, r_body, 0)

    scale = jnp.full((16,), jnp.float32(1.0 / N_TR), jnp.float32)

    def s_body(g, carry):
        base = g * 128
        for u in range(8):
            off = base + u * 16
            sacc_v[pl.ds(off, 16)] = sacc_v[pl.ds(off, 16)] * scale
        return carry

    lax.fori_loop(0, SGROUPS // 8, s_body, 0)
    pltpu.sync_copy(sacc_v, out_hbm.at[pl.ds(c * NPIX_PAD + sbase, STRIPE)])


_sc_kernels = {n: _make_das_sc(n // NW) for n in set(STAGES)}


def kernel(sinogram, v0, d_delay, ring_error):
    angle = (2.0 * jnp.pi / N_TR) * (jnp.arange(N_TR, dtype=jnp.float32) + 1.0)
    x_t = R_RING * jnp.cos(angle - jnp.pi)
    y_t = R_RING * jnp.sin(angle - jnp.pi)
    p = jnp.arange(NPIX_PAD, dtype=jnp.int32)
    xmap = (jnp.float32(-0.02)
            + jnp.float32(0.0002) * (p // GRID).astype(jnp.float32))
    ymap = (jnp.float32(-0.02)
            + jnp.float32(0.0002) * (p % GRID).astype(jnp.float32))
    xmap = xmap.reshape(BROWS, 128)
    ymap = ymap.reshape(BROWS, 128)
    scal = jnp.concatenate([
        v0.astype(jnp.float32),
        d_delay.astype(jnp.float32),
        ring_error.astype(jnp.float32),
        jnp.zeros((13,), jnp.float32),
    ])
    sino_flat = sinogram.reshape(-1)
    out = None
    toff = 0
    for n_tr_s in STAGES:
        xt_h = lax.slice(x_t, (toff,), (toff + n_tr_s,))
        yt_h = lax.slice(y_t, (toff,), (toff + n_tr_s,))
        idx_h = _compute_indices(xt_h, yt_h, scal, xmap, ymap,
                                 n_tr_s).reshape(-1)
        sino_h = lax.slice(sino_flat, (toff * N_TIME,),
                           ((toff + n_tr_s) * N_TIME,))
        part = _sc_kernels[n_tr_s](sino_h, idx_h)
        stage = part[:NPIX_PAD] + part[NPIX_PAD:]
        out = stage if out is None else out + stage
        toff += n_tr_s
    return out[:NPIX].reshape(GRID, GRID)
